# baseline (speedup 1.0000x reference)
import functools

import jax
import jax.numpy as jnp
from jax import lax
from jax.experimental import pallas as pl
from jax.experimental.pallas import tpu as pltpu
from jax.experimental.pallas import tpu_sc as plsc

N = 100000
D = 128
G = 512
CH = 3200
NW = 32
NPAD = CH * NW
NVEC = CH // 16
BLK = 51200
NBLK = NPAD // BLK


def _tc1_body(x_ref, w1_ref, b1_ref, wc_ref, y_ref):
    xb = x_ref[...].astype(jnp.bfloat16)
    h = jnp.dot(xb, w1_ref[...], preferred_element_type=jnp.float32)
    h = jnp.maximum(h.astype(jnp.bfloat16) + b1_ref[...], jnp.bfloat16(0.0))
    yT = lax.dot_general(wc_ref[...], h, (((0,), (1,)), ((), ())),
                         preferred_element_type=jnp.float32)
    y_ref[...] = yT


def _sc_body(y_hbm, ids_hbm, out_hbm, ids_v, y0_v, y1_v, a0, a1, ac):
    wid = lax.axis_index("s") * 2 + lax.axis_index("c")
    base = wid * CH

    pltpu.sync_copy(ids_hbm.at[pl.ds(base, CH)], ids_v)
    pltpu.sync_copy(y_hbm.at[pl.ds(base, CH)], y0_v)
    pltpu.sync_copy(y_hbm.at[pl.ds(NPAD + base, CH)], y1_v)

    zeros16 = jnp.zeros((16,), jnp.float32)
    for i in range(G // 16):
        a0[pl.ds(i * 16, 16)] = zeros16
        a1[pl.ds(i * 16, 16)] = zeros16
        ac[pl.ds(i * 16, 16)] = zeros16

    iota = lax.broadcasted_iota(jnp.int32, (16,), 0)
    ones16 = jnp.ones((16,), jnp.float32)

    def _step(it, carry):
        off = it * 16
        ids16 = ids_v[pl.ds(off, 16)]
        mask = (base + off + iota) < N
        plsc.addupdate_scatter(a0, [ids16], y0_v[pl.ds(off, 16)], mask=mask)
        plsc.addupdate_scatter(a1, [ids16], y1_v[pl.ds(off, 16)], mask=mask)
        plsc.addupdate_scatter(ac, [ids16], ones16, mask=mask)
        return carry

    lax.fori_loop(0, NVEC, _step, 0)

    pltpu.sync_copy(a0, out_hbm.at[pl.ds(wid * 3 * G, G)])
    pltpu.sync_copy(a1, out_hbm.at[pl.ds(wid * 3 * G + G, G)])
    pltpu.sync_copy(ac, out_hbm.at[pl.ds(wid * 3 * G + 2 * G, G)])


def _tc2_body(p_ref, bc_ref, bc0_ref, out_ref):
    a = jnp.sum(p_ref[...], axis=0)
    c = a[2:3, :]
    z = a[0:2, :] / jnp.maximum(c, 1.0) + bc_ref[...]
    z = jnp.where(c > 0.0, z, bc0_ref[...])
    zmax = jnp.max(z, axis=1, keepdims=True)
    e = jnp.exp(z - zmax)
    out_ref[...] = e / jnp.sum(e, axis=1, keepdims=True)


def kernel(x, batch, W1, b1, W2, b2, Wl1, bl1, Wl2, bl2):
    ids = jnp.zeros((NPAD,), jnp.int32).at[:N].set(batch.astype(jnp.int32))
    Wm = Wl1 @ Wl2
    Wc = (W2 @ Wm).astype(jnp.bfloat16)
    bc0 = bl1 @ Wl2 + bl2
    bc = (b2 @ Wm + bc0).reshape(2, 1)
    bc0 = bc0.reshape(2, 1)
    b1_2d = b1.reshape(1, 64).astype(jnp.bfloat16)
    W1 = W1.astype(jnp.bfloat16)

    y = pl.pallas_call(
        _tc1_body,
        grid=(NBLK,),
        in_specs=[
            pl.BlockSpec((BLK, D), lambda i: (i, 0)),
            pl.BlockSpec((D, 64), lambda i: (0, 0)),
            pl.BlockSpec((1, 64), lambda i: (0, 0)),
            pl.BlockSpec((64, 2), lambda i: (0, 0)),
        ],
        out_specs=pl.BlockSpec((2, BLK), lambda i: (0, i)),
        out_shape=jax.ShapeDtypeStruct((2, NPAD), jnp.float32),
    )(x, W1, b1_2d, Wc)

    partials = pl.kernel(
        _sc_body,
        mesh=plsc.VectorSubcoreMesh(core_axis_name="c", subcore_axis_name="s"),
        compiler_params=pltpu.CompilerParams(needs_layout_passes=False),
        out_type=jax.ShapeDtypeStruct((NW * 3 * G,), jnp.float32),
        scratch_types=[
            pltpu.VMEM((CH,), jnp.int32),
            pltpu.VMEM((CH,), jnp.float32),
            pltpu.VMEM((CH,), jnp.float32),
            pltpu.VMEM((G,), jnp.float32),
            pltpu.VMEM((G,), jnp.float32),
            pltpu.VMEM((G,), jnp.float32),
        ],
    )(y.reshape(2 * NPAD), ids)

    out = pl.pallas_call(
        _tc2_body,
        in_specs=[
            pl.BlockSpec((NW, 3, G), lambda: (0, 0, 0)),
            pl.BlockSpec((2, 1), lambda: (0, 0)),
            pl.BlockSpec((2, 1), lambda: (0, 0)),
        ],
        out_specs=pl.BlockSpec((2, G), lambda: (0, 0)),
        out_shape=jax.ShapeDtypeStruct((2, G), jnp.float32),
    )(partials.reshape(NW, 3, G), bc, bc0)
    return out.T


# speedup vs baseline: 1.0348x; 1.0348x over previous
import functools

import jax
import jax.numpy as jnp
from jax import lax
from jax.experimental import pallas as pl
from jax.experimental.pallas import tpu as pltpu
from jax.experimental.pallas import tpu_sc as plsc

N = 100000
D = 128
G = 512
CH = 3200
NW = 32
NPAD = CH * NW
NVEC = CH // 16
BLK = 25600
NBLK = NPAD // BLK


def _tc1_body(x_ref, w1_ref, b1_ref, wc_ref, y_ref):
    xb = x_ref[...].astype(jnp.bfloat16)
    h = jnp.dot(xb, w1_ref[...], preferred_element_type=jnp.float32)
    h = jnp.maximum(h.astype(jnp.bfloat16) + b1_ref[...], jnp.bfloat16(0.0))
    yT = lax.dot_general(wc_ref[...], h, (((0,), (1,)), ((), ())),
                         preferred_element_type=jnp.float32)
    y_ref[...] = yT


def _sc_body(y_hbm, ids_hbm, out_hbm, ids_v, y0_v, y1_v, a0, a1, ac):
    wid = lax.axis_index("s") * 2 + lax.axis_index("c")
    base = wid * CH

    pltpu.sync_copy(ids_hbm.at[pl.ds(base, CH)], ids_v)
    pltpu.sync_copy(y_hbm.at[pl.ds(base, CH)], y0_v)
    pltpu.sync_copy(y_hbm.at[pl.ds(NPAD + base, CH)], y1_v)

    zeros16 = jnp.zeros((16,), jnp.float32)
    for i in range(G // 16):
        a0[pl.ds(i * 16, 16)] = zeros16
        a1[pl.ds(i * 16, 16)] = zeros16
        ac[pl.ds(i * 16, 16)] = zeros16

    ones16 = jnp.ones((16,), jnp.float32)

    nvec = jnp.where(wid == NW - 1, (N - (NW - 1) * CH) // 16, NVEC)

    def _step(it, carry):
        off = it * 16
        ids16 = ids_v[pl.ds(off, 16)]
        plsc.addupdate_scatter(a0, [ids16], y0_v[pl.ds(off, 16)])
        plsc.addupdate_scatter(a1, [ids16], y1_v[pl.ds(off, 16)])
        plsc.addupdate_scatter(ac, [ids16], ones16)
        return carry

    lax.fori_loop(0, nvec, _step, 0)

    pltpu.sync_copy(a0, out_hbm.at[pl.ds(wid * 3 * G, G)])
    pltpu.sync_copy(a1, out_hbm.at[pl.ds(wid * 3 * G + G, G)])
    pltpu.sync_copy(ac, out_hbm.at[pl.ds(wid * 3 * G + 2 * G, G)])


def _tc2_body(p_ref, bc_ref, bc0_ref, out_ref):
    a = jnp.sum(p_ref[...], axis=0)
    c = a[2:3, :]
    z = a[0:2, :] / jnp.maximum(c, 1.0) + bc_ref[...]
    z = jnp.where(c > 0.0, z, bc0_ref[...])
    zmax = jnp.max(z, axis=1, keepdims=True)
    e = jnp.exp(z - zmax)
    out_ref[...] = e / jnp.sum(e, axis=1, keepdims=True)


def kernel(x, batch, W1, b1, W2, b2, Wl1, bl1, Wl2, bl2):
    ids = jnp.zeros((NPAD,), jnp.int32).at[:N].set(batch.astype(jnp.int32))
    Wm = Wl1 @ Wl2
    Wc = (W2 @ Wm).astype(jnp.bfloat16)
    bc0 = bl1 @ Wl2 + bl2
    bc = (b2 @ Wm + bc0).reshape(2, 1)
    bc0 = bc0.reshape(2, 1)
    b1_2d = b1.reshape(1, 64).astype(jnp.bfloat16)
    W1 = W1.astype(jnp.bfloat16)

    y = pl.pallas_call(
        _tc1_body,
        grid=(NBLK,),
        in_specs=[
            pl.BlockSpec((BLK, D), lambda i: (i, 0)),
            pl.BlockSpec((D, 64), lambda i: (0, 0)),
            pl.BlockSpec((1, 64), lambda i: (0, 0)),
            pl.BlockSpec((64, 2), lambda i: (0, 0)),
        ],
        out_specs=pl.BlockSpec((2, BLK), lambda i: (0, i)),
        out_shape=jax.ShapeDtypeStruct((2, NPAD), jnp.float32),
    )(x, W1, b1_2d, Wc)

    partials = pl.kernel(
        _sc_body,
        mesh=plsc.VectorSubcoreMesh(core_axis_name="c", subcore_axis_name="s"),
        compiler_params=pltpu.CompilerParams(needs_layout_passes=False),
        out_type=jax.ShapeDtypeStruct((NW * 3 * G,), jnp.float32),
        scratch_types=[
            pltpu.VMEM((CH,), jnp.int32),
            pltpu.VMEM((CH,), jnp.float32),
            pltpu.VMEM((CH,), jnp.float32),
            pltpu.VMEM((G,), jnp.float32),
            pltpu.VMEM((G,), jnp.float32),
            pltpu.VMEM((G,), jnp.float32),
        ],
    )(y.reshape(2 * NPAD), ids)

    out = pl.pallas_call(
        _tc2_body,
        in_specs=[
            pl.BlockSpec((NW, 3, G), lambda: (0, 0, 0)),
            pl.BlockSpec((2, 1), lambda: (0, 0)),
            pl.BlockSpec((2, 1), lambda: (0, 0)),
        ],
        out_specs=pl.BlockSpec((2, G), lambda: (0, 0)),
        out_shape=jax.ShapeDtypeStruct((2, G), jnp.float32),
    )(partials.reshape(NW, 3, G), bc, bc0)
    return out.T


# speedup vs baseline: 1.0483x; 1.0130x over previous
import functools

import jax
import jax.numpy as jnp
from jax import lax
from jax.experimental import pallas as pl
from jax.experimental.pallas import tpu as pltpu
from jax.experimental.pallas import tpu_sc as plsc

N = 100000
D = 128
G = 512
CH = 3200
NW = 32
NPAD = CH * NW
NVEC = CH // 16
BLK = 25600
NBLK = NPAD // BLK


def _tc1_body(x_ref, w1_ref, b1_ref, wc_ref, y_ref):
    xb = x_ref[...].astype(jnp.bfloat16)
    h = jnp.dot(xb, w1_ref[...], preferred_element_type=jnp.float32)
    h = jnp.maximum(h.astype(jnp.bfloat16) + b1_ref[...], jnp.bfloat16(0.0))
    yT = lax.dot_general(wc_ref[...], h, (((0,), (1,)), ((), ())),
                         preferred_element_type=jnp.float32)
    y_ref[...] = yT


def _sc_body(y_hbm, ids_hbm, out_hbm, ids_v, y0_v, y1_v, a0, a1, ac):
    wid = lax.axis_index("s") * 2 + lax.axis_index("c")
    base = wid * CH

    pltpu.sync_copy(ids_hbm.at[pl.ds(base, CH)], ids_v)
    pltpu.sync_copy(y_hbm.at[pl.ds(base, CH)], y0_v)
    pltpu.sync_copy(y_hbm.at[pl.ds(NPAD + base, CH)], y1_v)

    zeros16 = jnp.zeros((16,), jnp.float32)
    for i in range(G // 16):
        a0[pl.ds(i * 16, 16)] = zeros16
        a1[pl.ds(i * 16, 16)] = zeros16
        ac[pl.ds(i * 16, 16)] = zeros16

    ones16 = jnp.ones((16,), jnp.float32)

    nvec = jnp.where(wid == NW - 1, (N - (NW - 1) * CH) // 16, NVEC)

    @plsc.parallel_loop(0, nvec * 16, 16, unroll=4)
    def _step(off):
        ids16 = ids_v[pl.ds(off, 16)]
        plsc.addupdate_scatter(a0, [ids16], y0_v[pl.ds(off, 16)])
        plsc.addupdate_scatter(a1, [ids16], y1_v[pl.ds(off, 16)])
        plsc.addupdate_scatter(ac, [ids16], ones16)

    pltpu.sync_copy(a0, out_hbm.at[pl.ds(wid * 3 * G, G)])
    pltpu.sync_copy(a1, out_hbm.at[pl.ds(wid * 3 * G + G, G)])
    pltpu.sync_copy(ac, out_hbm.at[pl.ds(wid * 3 * G + 2 * G, G)])


def _tc2_body(p_ref, bc_ref, bc0_ref, out_ref):
    a = jnp.sum(p_ref[...], axis=0)
    c = a[2:3, :]
    z = a[0:2, :] / jnp.maximum(c, 1.0) + bc_ref[...]
    z = jnp.where(c > 0.0, z, bc0_ref[...])
    zmax = jnp.max(z, axis=1, keepdims=True)
    e = jnp.exp(z - zmax)
    out_ref[...] = e / jnp.sum(e, axis=1, keepdims=True)


def kernel(x, batch, W1, b1, W2, b2, Wl1, bl1, Wl2, bl2):
    ids = jnp.zeros((NPAD,), jnp.int32).at[:N].set(batch.astype(jnp.int32))
    Wm = Wl1 @ Wl2
    Wc = (W2 @ Wm).astype(jnp.bfloat16)
    bc0 = bl1 @ Wl2 + bl2
    bc = (b2 @ Wm + bc0).reshape(2, 1)
    bc0 = bc0.reshape(2, 1)
    b1_2d = b1.reshape(1, 64).astype(jnp.bfloat16)
    W1 = W1.astype(jnp.bfloat16)

    y = pl.pallas_call(
        _tc1_body,
        grid=(NBLK,),
        in_specs=[
            pl.BlockSpec((BLK, D), lambda i: (i, 0)),
            pl.BlockSpec((D, 64), lambda i: (0, 0)),
            pl.BlockSpec((1, 64), lambda i: (0, 0)),
            pl.BlockSpec((64, 2), lambda i: (0, 0)),
        ],
        out_specs=pl.BlockSpec((2, BLK), lambda i: (0, i)),
        out_shape=jax.ShapeDtypeStruct((2, NPAD), jnp.float32),
    )(x, W1, b1_2d, Wc)

    partials = pl.kernel(
        _sc_body,
        mesh=plsc.VectorSubcoreMesh(core_axis_name="c", subcore_axis_name="s"),
        compiler_params=pltpu.CompilerParams(needs_layout_passes=False),
        out_type=jax.ShapeDtypeStruct((NW * 3 * G,), jnp.float32),
        scratch_types=[
            pltpu.VMEM((CH,), jnp.int32),
            pltpu.VMEM((CH,), jnp.float32),
            pltpu.VMEM((CH,), jnp.float32),
            pltpu.VMEM((G,), jnp.float32),
            pltpu.VMEM((G,), jnp.float32),
            pltpu.VMEM((G,), jnp.float32),
        ],
    )(y.reshape(2 * NPAD), ids)

    out = pl.pallas_call(
        _tc2_body,
        in_specs=[
            pl.BlockSpec((NW, 3, G), lambda: (0, 0, 0)),
            pl.BlockSpec((2, 1), lambda: (0, 0)),
            pl.BlockSpec((2, 1), lambda: (0, 0)),
        ],
        out_specs=pl.BlockSpec((2, G), lambda: (0, 0)),
        out_shape=jax.ShapeDtypeStruct((2, G), jnp.float32),
    )(partials.reshape(NW, 3, G), bc, bc0)
    return out.T
